# Initial kernel scaffold; baseline (speedup 1.0000x reference)
import jax
import jax.numpy as jnp
from jax.experimental import pallas as pl


def kernel(atom_features, bond_features, pair_indices, kernel, bias):
    raise NotImplementedError("write your pallas kernel here")


# speedup vs baseline: 1.9532x; 1.9532x over previous
import functools

import jax
import jax.numpy as jnp
from jax import lax
from jax.experimental import pallas as pl
from jax.experimental.pallas import tpu as pltpu
from jax.experimental.pallas import tpu_sc as plsc

_NN = 50000
_NE = 100000
_AD = 32
_BD = 16
_NW = 32
_CH = 128
_E2 = 102400
_SP_ROWS = 51200


def _sc_gather(atom, nbr_pad):
    per_w = _E2 // _NW
    n_chunks = per_w // _CH
    mesh = plsc.VectorSubcoreMesh(core_axis_name="c", subcore_axis_name="s")

    @functools.partial(
        pl.kernel,
        mesh=mesh,
        out_type=jax.ShapeDtypeStruct((_E2, _AD), jnp.float32),
        scratch_types=[
            pltpu.VMEM((_CH,), jnp.int32),
            pltpu.VMEM((_CH, _AD), jnp.float32),
            pltpu.SemaphoreType.DMA,
        ],
        compiler_params=pltpu.CompilerParams(use_tc_tiling_on_sc=False),
    )
    def k(atom_hbm, idx_hbm, out_hbm, idxc, rows, sem):
        wid = lax.axis_index("s") * 2 + lax.axis_index("c")
        base = wid * per_w

        def body(c, carry):
            off = base + c * _CH
            pltpu.sync_copy(idx_hbm.at[pl.ds(off, _CH)], idxc)
            pltpu.async_copy(atom_hbm.at[idxc], rows, sem).wait()
            pltpu.sync_copy(rows, out_hbm.at[pl.ds(off, _CH)])
            return carry

        lax.fori_loop(0, n_chunks, body, 0)

    return k(atom, nbr_pad)


def _tc_transform(bond_pad, a_nbr, wcat):
    be = 512

    def body(bond_ref, a_ref, w_ref, out_ref):
        a = a_ref[...]
        b = bond_ref[...]
        h = jnp.dot(a, w_ref[...], preferred_element_type=jnp.float32)
        acc = h[:, 512:544]
        for kk in range(_BD):
            acc = acc + b[:, kk:kk + 1] * h[:, kk * 32:(kk + 1) * 32]
        out_ref[...] = acc

    return pl.pallas_call(
        body,
        grid=(_E2 // be,),
        in_specs=[
            pl.BlockSpec((be, _BD), lambda i: (i, 0)),
            pl.BlockSpec((be, _AD), lambda i: (i, 0)),
            pl.BlockSpec((_AD, 544), lambda i: (0, 0)),
        ],
        out_specs=pl.BlockSpec((be, _AD), lambda i: (i, 0)),
        out_shape=jax.ShapeDtypeStruct((_E2, _AD), jnp.float32),
    )(bond_pad, a_nbr, wcat)


def _sc_segment_sum(transformed, src_pad):
    per_t = _E2 // 16
    n_chunks = per_t // _CH
    zrows = _SP_ROWS // 16
    orows = _NN // 16
    mesh = plsc.VectorSubcoreMesh(core_axis_name="c", subcore_axis_name="s")

    @functools.partial(
        pl.kernel,
        mesh=mesh,
        out_type=jax.ShapeDtypeStruct((_NN, _AD), jnp.float32),
        scratch_types=[
            pltpu.VMEM((_CH,), jnp.int32),
            pltpu.VMEM((_CH, _AD), jnp.float32),
            pltpu.VMEM((_CH, _AD), jnp.float32),
            pltpu.VMEM_SHARED((_SP_ROWS, _AD), jnp.float32),
            pltpu.SemaphoreType.DMA,
        ],
        compiler_params=pltpu.CompilerParams(use_tc_tiling_on_sc=False),
    )
    def k(t_hbm, src_hbm, out_hbm, idxc, rows, zbuf, acc_sp, sem):
        cid = lax.axis_index("c")
        sid = lax.axis_index("s")

        @pl.when(cid == 0)
        def _zero():
            zero16 = jnp.zeros((16,), jnp.float32)

            def zb(r, carry):
                zbuf[r, 0:16] = zero16
                zbuf[r, 16:32] = zero16
                return carry

            lax.fori_loop(0, _CH, zb, 0)

            def zs(cnk, carry):
                pltpu.sync_copy(
                    zbuf, acc_sp.at[pl.ds(sid * zrows + cnk * _CH, _CH)])
                return carry

            lax.fori_loop(0, zrows // _CH, zs, 0)

        plsc.subcore_barrier()

        @pl.when(cid == 0)
        def _scatter():
            base = sid * per_t

            def body(cnk, carry):
                off = base + cnk * _CH
                pltpu.sync_copy(src_hbm.at[pl.ds(off, _CH)], idxc)
                pltpu.sync_copy(t_hbm.at[pl.ds(off, _CH)], rows)
                pltpu.sync_copy(rows, acc_sp.at[idxc], add=True)
                return carry

            lax.fori_loop(0, n_chunks, body, 0)

        plsc.subcore_barrier()

        @pl.when(cid == 0)
        def _flush():
            pltpu.sync_copy(acc_sp.at[pl.ds(sid * orows, orows)],
                            out_hbm.at[pl.ds(sid * orows, orows)])

    return k(transformed, src_pad)


def kernel(atom_features, bond_features, pair_indices, kernel, bias):
    w_aug = jnp.concatenate([kernel, bias[None, :]], axis=0)
    wcat = w_aug.reshape(17, _AD, _AD).transpose(2, 0, 1).reshape(_AD, 17 * _AD)

    pad = _E2 - _NE
    nbr_pad = jnp.concatenate(
        [pair_indices[:, 1], jnp.zeros((pad,), jnp.int32)])
    src_pad = jnp.concatenate(
        [pair_indices[:, 0], jnp.full((pad,), _NN, jnp.int32)])
    bond_pad = jnp.concatenate(
        [bond_features, jnp.zeros((pad, _BD), jnp.float32)])

    a_nbr = _sc_gather(atom_features, nbr_pad)
    transformed = _tc_transform(bond_pad, a_nbr, wcat)
    return _sc_segment_sum(transformed, src_pad)


# speedup vs baseline: 3.1448x; 1.6101x over previous
import functools

import jax
import jax.numpy as jnp
from jax import lax
from jax.experimental import pallas as pl
from jax.experimental.pallas import tpu as pltpu
from jax.experimental.pallas import tpu_sc as plsc

_NN = 50000
_NE = 100000
_AD = 32
_BD = 16
_NW = 32
_CH = 128
_E2 = 102400
_SP_ROWS = 51200


def _sc_gather(atom, nbr_pad):
    per_w = _E2 // _NW
    n_chunks = per_w // _CH
    mesh = plsc.VectorSubcoreMesh(core_axis_name="c", subcore_axis_name="s")

    @functools.partial(
        pl.kernel,
        mesh=mesh,
        out_type=jax.ShapeDtypeStruct((_E2, _AD), jnp.float32),
        scratch_types=[
            pltpu.VMEM((_CH,), jnp.int32),
            pltpu.VMEM((_CH, _AD), jnp.float32),
            pltpu.SemaphoreType.DMA,
        ],
        compiler_params=pltpu.CompilerParams(use_tc_tiling_on_sc=False),
    )
    def k(atom_hbm, idx_hbm, out_hbm, idxc, rows, sem):
        wid = lax.axis_index("s") * 2 + lax.axis_index("c")
        base = wid * per_w

        def body(c, carry):
            off = base + c * _CH
            pltpu.sync_copy(idx_hbm.at[pl.ds(off, _CH)], idxc)
            pltpu.async_copy(atom_hbm.at[idxc], rows, sem).wait()
            pltpu.sync_copy(rows, out_hbm.at[pl.ds(off, _CH)])
            return carry

        lax.fori_loop(0, n_chunks, body, 0)

    return k(atom, nbr_pad)


def _tc_transform(bond_t, a_nbr, wcat_t):
    be = 512

    def body(bt_ref, a_ref, wt_ref, out_ref):
        a = a_ref[...]
        ht = lax.dot_general(wt_ref[...], a, (((1,), (1,)), ((), ())),
                             preferred_element_type=jnp.float32)
        bt = bt_ref[...]
        acc = ht[512:544, :]
        for kk in range(_BD):
            acc = acc + bt[kk:kk + 1, :] * ht[kk * 32:(kk + 1) * 32, :]
        out_ref[...] = acc.T

    return pl.pallas_call(
        body,
        grid=(_E2 // be,),
        in_specs=[
            pl.BlockSpec((_BD, be), lambda i: (0, i)),
            pl.BlockSpec((be, _AD), lambda i: (i, 0)),
            pl.BlockSpec((544, _AD), lambda i: (0, 0)),
        ],
        out_specs=pl.BlockSpec((be, _AD), lambda i: (i, 0)),
        out_shape=jax.ShapeDtypeStruct((_E2, _AD), jnp.float32),
    )(bond_t, a_nbr, wcat_t)


def _sc_segment_sum(transformed, src_pad):
    per_t = _E2 // 16
    n_chunks = per_t // _CH
    zrows = _SP_ROWS // 16
    orows = _NN // 16
    mesh = plsc.VectorSubcoreMesh(core_axis_name="c", subcore_axis_name="s")

    @functools.partial(
        pl.kernel,
        mesh=mesh,
        out_type=jax.ShapeDtypeStruct((_NN, _AD), jnp.float32),
        scratch_types=[
            pltpu.VMEM((_CH,), jnp.int32),
            pltpu.VMEM((_CH, _AD), jnp.float32),
            pltpu.VMEM((_CH, _AD), jnp.float32),
            pltpu.VMEM_SHARED((_SP_ROWS, _AD), jnp.float32),
            pltpu.SemaphoreType.DMA,
        ],
        compiler_params=pltpu.CompilerParams(use_tc_tiling_on_sc=False),
    )
    def k(t_hbm, src_hbm, out_hbm, idxc, rows, zbuf, acc_sp, sem):
        cid = lax.axis_index("c")
        sid = lax.axis_index("s")

        @pl.when(cid == 0)
        def _zero():
            zero16 = jnp.zeros((16,), jnp.float32)

            def zb(r, carry):
                zbuf[r, 0:16] = zero16
                zbuf[r, 16:32] = zero16
                return carry

            lax.fori_loop(0, _CH, zb, 0)

            def zs(cnk, carry):
                pltpu.sync_copy(
                    zbuf, acc_sp.at[pl.ds(sid * zrows + cnk * _CH, _CH)])
                return carry

            lax.fori_loop(0, zrows // _CH, zs, 0)

        plsc.subcore_barrier()

        @pl.when(cid == 0)
        def _scatter():
            base = sid * per_t

            def body(cnk, carry):
                off = base + cnk * _CH
                pltpu.sync_copy(src_hbm.at[pl.ds(off, _CH)], idxc)
                pltpu.sync_copy(t_hbm.at[pl.ds(off, _CH)], rows)
                pltpu.sync_copy(rows, acc_sp.at[idxc], add=True)
                return carry

            lax.fori_loop(0, n_chunks, body, 0)

        plsc.subcore_barrier()

        @pl.when(cid == 0)
        def _flush():
            pltpu.sync_copy(acc_sp.at[pl.ds(sid * orows, orows)],
                            out_hbm.at[pl.ds(sid * orows, orows)])

    return k(transformed, src_pad)


def kernel(atom_features, bond_features, pair_indices, kernel, bias):
    w_aug = jnp.concatenate([kernel, bias[None, :]], axis=0)
    wcat_t = w_aug.reshape(17 * _AD, _AD)

    pad = _E2 - _NE
    nbr_pad = jnp.concatenate(
        [pair_indices[:, 1], jnp.zeros((pad,), jnp.int32)])
    src_pad = jnp.concatenate(
        [pair_indices[:, 0], jnp.full((pad,), _NN, jnp.int32)])
    bond_t = jnp.concatenate(
        [bond_features, jnp.zeros((pad, _BD), jnp.float32)]).T

    a_nbr = _sc_gather(atom_features, nbr_pad)
    transformed = _tc_transform(bond_t, a_nbr, wcat_t)
    return _sc_segment_sum(transformed, src_pad)


# speedup vs baseline: 4.2475x; 1.3507x over previous
import functools

import jax
import jax.numpy as jnp
from jax import lax
from jax.experimental import pallas as pl
from jax.experimental.pallas import tpu as pltpu
from jax.experimental.pallas import tpu_sc as plsc

_NN = 50000
_NE = 100000
_AD = 32
_BD = 16
_NW = 32
_CH = 128
_E2 = 102400
_SP_ROWS = 51200


_GRP = 5


def _sc_gather(atom, nbr_pad):
    per_w = _E2 // _NW
    n_grp = per_w // (_CH * _GRP)
    mesh = plsc.VectorSubcoreMesh(core_axis_name="c", subcore_axis_name="s")

    @functools.partial(
        pl.kernel,
        mesh=mesh,
        out_type=jax.ShapeDtypeStruct((_E2, _AD), jnp.float32),
        scratch_types=[
            pltpu.VMEM((per_w,), jnp.int32),
            pltpu.VMEM((_GRP, _CH, _AD), jnp.float32),
            pltpu.SemaphoreType.DMA,
            pltpu.SemaphoreType.DMA,
        ],
        compiler_params=pltpu.CompilerParams(use_tc_tiling_on_sc=False),
    )
    def k(atom_hbm, idx_hbm, out_hbm, idx_all, bufs, gsem, wsem):
        wid = lax.axis_index("s") * 2 + lax.axis_index("c")
        base = wid * per_w
        pltpu.sync_copy(idx_hbm.at[pl.ds(base, per_w)], idx_all)

        def group(g, carry):
            cbase = g * _GRP
            hs = [
                pltpu.async_copy(
                    atom_hbm.at[idx_all.at[pl.ds((cbase + b) * _CH, _CH)]],
                    bufs.at[b], gsem)
                for b in range(_GRP)
            ]
            for h in hs:
                h.wait()
            ws = [
                pltpu.async_copy(
                    bufs.at[b],
                    out_hbm.at[pl.ds(base + (cbase + b) * _CH, _CH)], wsem)
                for b in range(_GRP)
            ]
            for w in ws:
                w.wait()
            return carry

        lax.fori_loop(0, n_grp, group, 0)

    return k(atom, nbr_pad)


def _tc_transform(bond_t, a_nbr, wcat_t):
    be = 1024

    def body(bt_ref, a_ref, wt_ref, out_ref):
        a = a_ref[...]
        ht = lax.dot_general(wt_ref[...], a, (((1,), (1,)), ((), ())),
                             preferred_element_type=jnp.float32)
        bt = bt_ref[...]
        acc = ht[512:544, :]
        for kk in range(_BD):
            acc = acc + bt[kk:kk + 1, :] * ht[kk * 32:(kk + 1) * 32, :]
        out_ref[...] = acc.T

    return pl.pallas_call(
        body,
        grid=(_E2 // be,),
        in_specs=[
            pl.BlockSpec((_BD, be), lambda i: (0, i)),
            pl.BlockSpec((be, _AD), lambda i: (i, 0)),
            pl.BlockSpec((544, _AD), lambda i: (0, 0)),
        ],
        out_specs=pl.BlockSpec((be, _AD), lambda i: (i, 0)),
        out_shape=jax.ShapeDtypeStruct((_E2, _AD), jnp.float32),
    )(bond_t, a_nbr, wcat_t)


def _sc_segment_sum(transformed, src2):
    per_t = _E2 // 16
    n_grp = per_t // (_CH * _GRP)
    zrows = _SP_ROWS // 16
    orows = _NN // 16
    mesh = plsc.VectorSubcoreMesh(core_axis_name="c", subcore_axis_name="s")

    @functools.partial(
        pl.kernel,
        mesh=mesh,
        out_type=jax.ShapeDtypeStruct((_NN, _AD), jnp.float32),
        scratch_types=[
            pltpu.VMEM((_GRP, _CH), jnp.int32),
            pltpu.VMEM((_GRP, _CH, _AD), jnp.float32),
            pltpu.VMEM((_CH, _AD), jnp.float32),
            pltpu.VMEM_SHARED((_SP_ROWS, _AD), jnp.float32),
            pltpu.SemaphoreType.DMA,
            pltpu.SemaphoreType.DMA,
        ],
        compiler_params=pltpu.CompilerParams(use_tc_tiling_on_sc=False),
    )
    def k(t_hbm, src_hbm, out_hbm, idx_all, bufs, zbuf, acc_sp, lsem, ssem):
        cid = lax.axis_index("c")
        sid = lax.axis_index("s")

        @pl.when(cid == 0)
        def _zero():
            zero16 = jnp.zeros((16,), jnp.float32)

            def zb(r, carry):
                zbuf[r, 0:16] = zero16
                zbuf[r, 16:32] = zero16
                return carry

            lax.fori_loop(0, _CH, zb, 0)

            def zs(cnk, carry):
                pltpu.sync_copy(
                    zbuf, acc_sp.at[pl.ds(sid * zrows + cnk * _CH, _CH)])
                return carry

            lax.fori_loop(0, zrows // _CH, zs, 0)

        plsc.subcore_barrier()

        @pl.when(cid == 0)
        def _scatter():
            base = sid * per_t

            def group(g, carry):
                cbase = g * _GRP
                pltpu.sync_copy(
                    src_hbm.at[pl.ds(sid * (per_t // _CH) + cbase, _GRP)],
                    idx_all)
                hs = [
                    pltpu.async_copy(
                        t_hbm.at[pl.ds(base + (cbase + b) * _CH, _CH)],
                        bufs.at[b], lsem)
                    for b in range(_GRP)
                ]
                for h in hs:
                    h.wait()
                ws = [
                    pltpu.async_copy(
                        bufs.at[b], acc_sp.at[idx_all.at[b]],
                        ssem, add=True)
                    for b in range(_GRP)
                ]
                for w in ws:
                    w.wait()
                return carry

            lax.fori_loop(0, n_grp, group, 0)

        plsc.subcore_barrier()

        @pl.when(cid == 0)
        def _flush():
            pltpu.sync_copy(acc_sp.at[pl.ds(sid * orows, orows)],
                            out_hbm.at[pl.ds(sid * orows, orows)])

    return k(transformed, src2)


def kernel(atom_features, bond_features, pair_indices, kernel, bias):
    w_aug = jnp.concatenate([kernel, bias[None, :]], axis=0)
    wcat_t = w_aug.reshape(17 * _AD, _AD)

    pad = _E2 - _NE
    nbr_pad = jnp.concatenate(
        [pair_indices[:, 1], jnp.zeros((pad,), jnp.int32)])
    src2 = jnp.concatenate(
        [pair_indices[:, 0], jnp.full((pad,), _NN, jnp.int32)]
    ).reshape(_E2 // _CH, _CH)
    bond_t = jnp.concatenate(
        [bond_features, jnp.zeros((pad, _BD), jnp.float32)]).T

    a_nbr = _sc_gather(atom_features, nbr_pad)
    transformed = _tc_transform(bond_t, a_nbr, wcat_t)
    return _sc_segment_sum(transformed, src2)


# speedup vs baseline: 4.8134x; 1.1332x over previous
import functools

import jax
import jax.numpy as jnp
from jax import lax
from jax.experimental import pallas as pl
from jax.experimental.pallas import tpu as pltpu
from jax.experimental.pallas import tpu_sc as plsc

_NN = 50000
_NE = 100000
_AD = 32
_BD = 16
_NW = 32
_CH = 128
_E2 = 102400
_SP_ROWS = 51200


_GRP = 5


def _sc_gather(atom, nbr_pad):
    per_w = _E2 // _NW
    n_grp = per_w // (_CH * _GRP)
    mesh = plsc.VectorSubcoreMesh(core_axis_name="c", subcore_axis_name="s")

    @functools.partial(
        pl.kernel,
        mesh=mesh,
        out_type=jax.ShapeDtypeStruct((_E2, _AD), jnp.float32),
        scratch_types=[
            pltpu.VMEM((per_w,), jnp.int32),
            pltpu.VMEM((_GRP, _CH, _AD), jnp.float32),
            pltpu.SemaphoreType.DMA,
            pltpu.SemaphoreType.DMA,
        ],
        compiler_params=pltpu.CompilerParams(use_tc_tiling_on_sc=False),
    )
    def k(atom_hbm, idx_hbm, out_hbm, idx_all, bufs, gsem, wsem):
        wid = lax.axis_index("s") * 2 + lax.axis_index("c")
        base = wid * per_w
        pltpu.sync_copy(idx_hbm.at[pl.ds(base, per_w)], idx_all)

        def group(g, carry):
            cbase = g * _GRP
            hs = [
                pltpu.async_copy(
                    atom_hbm.at[idx_all.at[pl.ds((cbase + b) * _CH, _CH)]],
                    bufs.at[b], gsem)
                for b in range(_GRP)
            ]
            for h in hs:
                h.wait()
            ws = [
                pltpu.async_copy(
                    bufs.at[b],
                    out_hbm.at[pl.ds(base + (cbase + b) * _CH, _CH)], wsem)
                for b in range(_GRP)
            ]
            for w in ws:
                w.wait()
            return carry

        lax.fori_loop(0, n_grp, group, 0)

    return k(atom, nbr_pad)


def _tc_transform(bond_q, a_pk, wcat_t):
    be = 1024
    rb = be // 4

    def body(bq_ref, a_ref, wt_ref, out_ref):
        p = a_ref[...]
        bq = bq_ref[...]
        cols = []
        for g in range(4):
            pg = p[:, g * _AD:(g + 1) * _AD]
            ht = lax.dot_general(wt_ref[...], pg, (((1,), (1,)), ((), ())),
                                 preferred_element_type=jnp.float32)
            bt = bq[g]
            acc = ht[512:544, :]
            for kk in range(_BD):
                acc = acc + bt[kk:kk + 1, :] * ht[kk * 32:(kk + 1) * 32, :]
            cols.append(acc.T)
        out_ref[...] = jnp.concatenate(cols, axis=1)

    return pl.pallas_call(
        body,
        grid=(_E2 // be,),
        in_specs=[
            pl.BlockSpec((4, _BD, rb), lambda i: (0, 0, i)),
            pl.BlockSpec((rb, 4 * _AD), lambda i: (i, 0)),
            pl.BlockSpec((544, _AD), lambda i: (0, 0)),
        ],
        out_specs=pl.BlockSpec((rb, 4 * _AD), lambda i: (i, 0)),
        out_shape=jax.ShapeDtypeStruct((_E2 // 4, 4 * _AD), jnp.float32),
    )(bond_q, a_pk, wcat_t)


def _sc_segment_sum(transformed, src2):
    per_t = _E2 // 16
    n_grp = per_t // (_CH * _GRP)
    zrows = _SP_ROWS // 16
    orows = _NN // 16
    mesh = plsc.VectorSubcoreMesh(core_axis_name="c", subcore_axis_name="s")

    @functools.partial(
        pl.kernel,
        mesh=mesh,
        out_type=jax.ShapeDtypeStruct((_NN, _AD), jnp.float32),
        scratch_types=[
            pltpu.VMEM((_GRP, _CH), jnp.int32),
            pltpu.VMEM((_GRP, _CH, _AD), jnp.float32),
            pltpu.VMEM((_CH, _AD), jnp.float32),
            pltpu.VMEM_SHARED((_SP_ROWS, _AD), jnp.float32),
            pltpu.SemaphoreType.DMA,
            pltpu.SemaphoreType.DMA,
        ],
        compiler_params=pltpu.CompilerParams(use_tc_tiling_on_sc=False),
    )
    def k(t_hbm, src_hbm, out_hbm, idx_all, bufs, zbuf, acc_sp, lsem, ssem):
        cid = lax.axis_index("c")
        sid = lax.axis_index("s")

        @pl.when(cid == 0)
        def _zero():
            zero16 = jnp.zeros((16,), jnp.float32)

            def zb(r, carry):
                zbuf[r, 0:16] = zero16
                zbuf[r, 16:32] = zero16
                return carry

            lax.fori_loop(0, _CH, zb, 0)

            def zs(cnk, carry):
                pltpu.sync_copy(
                    zbuf, acc_sp.at[pl.ds(sid * zrows + cnk * _CH, _CH)])
                return carry

            lax.fori_loop(0, zrows // _CH, zs, 0)

        plsc.subcore_barrier()

        @pl.when(cid == 0)
        def _scatter():
            base = sid * per_t

            def group(g, carry):
                cbase = g * _GRP
                pltpu.sync_copy(
                    src_hbm.at[pl.ds(sid * (per_t // _CH) + cbase, _GRP)],
                    idx_all)
                hs = [
                    pltpu.async_copy(
                        t_hbm.at[pl.ds(base + (cbase + b) * _CH, _CH)],
                        bufs.at[b], lsem)
                    for b in range(_GRP)
                ]
                for h in hs:
                    h.wait()
                ws = [
                    pltpu.async_copy(
                        bufs.at[b], acc_sp.at[idx_all.at[b]],
                        ssem, add=True)
                    for b in range(_GRP)
                ]
                for w in ws:
                    w.wait()
                return carry

            lax.fori_loop(0, n_grp, group, 0)

        plsc.subcore_barrier()

        @pl.when(cid == 0)
        def _flush():
            pltpu.sync_copy(acc_sp.at[pl.ds(sid * orows, orows)],
                            out_hbm.at[pl.ds(sid * orows, orows)])

    return k(transformed, src2)


def kernel(atom_features, bond_features, pair_indices, kernel, bias):
    w_aug = jnp.concatenate([kernel, bias[None, :]], axis=0)
    wcat_t = w_aug.reshape(17 * _AD, _AD)

    pad = _E2 - _NE
    nbr_pad = jnp.concatenate(
        [pair_indices[:, 1], jnp.zeros((pad,), jnp.int32)])
    src2 = jnp.concatenate(
        [pair_indices[:, 0], jnp.full((pad,), _NN, jnp.int32)]
    ).reshape(_E2 // _CH, _CH)
    bond_q = jnp.concatenate(
        [bond_features, jnp.zeros((pad, _BD), jnp.float32)]
    ).reshape(_E2 // 4, 4, _BD).transpose(1, 2, 0)

    a_nbr = _sc_gather(atom_features, nbr_pad)
    a_pk = a_nbr.reshape(_E2 // 4, 4 * _AD)
    t_pk = _tc_transform(bond_q, a_pk, wcat_t)
    transformed = t_pk.reshape(_E2, _AD)
    return _sc_segment_sum(transformed, src2)
